# Initial kernel scaffold; baseline (speedup 1.0000x reference)
import jax
import jax.numpy as jnp
from jax.experimental import pallas as pl


def kernel(x, edge_index, W1, b1, W2, b2, Wd, bd):
    raise NotImplementedError("write your pallas kernel here")


# speedup vs baseline: 89.5596x; 89.5596x over previous
import functools

import jax
import jax.numpy as jnp
from jax import lax
from jax.experimental import pallas as pl
from jax.experimental.pallas import tpu as pltpu
from jax.experimental.pallas import tpu_sc as plsc

NC = 2
NS = 16
L = 16
NW = NC * NS


def _pad_up(v, m):
    return ((v + m - 1) // m) * m


def _sc_segsum(dst_pad, src_pad, u, NP):
    Ep = dst_pad.shape[0]
    EPT = Ep // NW
    gather = u is not None

    mesh = plsc.VectorSubcoreMesh(core_axis_name="c", subcore_axis_name="s",
                                  num_cores=NC)

    scratch = [
        pltpu.VMEM((EPT,), jnp.int32),
        pltpu.VMEM((NP,), jnp.float32),
    ]
    if gather:
        scratch += [
            pltpu.VMEM((EPT,), jnp.int32),
            pltpu.VMEM((NP,), jnp.float32),
        ]

    @functools.partial(
        pl.kernel,
        out_type=jax.ShapeDtypeStruct((NW * NP,), jnp.float32),
        mesh=mesh,
        scratch_types=scratch,
        compiler_params=pltpu.CompilerParams(needs_layout_passes=False),
    )
    def seg_kernel(*refs):
        if gather:
            dst_hbm, src_hbm, u_hbm, out_hbm, dst_v, acc_v, src_v, u_v = refs
        else:
            dst_hbm, out_hbm, dst_v, acc_v = refs
        cid = lax.axis_index("c")
        sid = lax.axis_index("s")
        wid = sid * NC + cid

        def zero_body(i, _):
            acc_v[pl.ds(i * L, L)] = jnp.zeros((L,), jnp.float32)
            return 0
        lax.fori_loop(0, NP // L, zero_body, 0)

        pltpu.sync_copy(dst_hbm.at[pl.ds(wid * EPT, EPT)], dst_v)
        if gather:
            pltpu.sync_copy(src_hbm.at[pl.ds(wid * EPT, EPT)], src_v)
            pltpu.sync_copy(u_hbm, u_v)

        ones = jnp.full((L,), 1.0, jnp.float32)

        def scat_body(i, _):
            d = dst_v[pl.ds(i * L, L)]
            if gather:
                s = src_v[pl.ds(i * L, L)]
                vals = plsc.load_gather(u_v, [s])
            else:
                vals = ones
            plsc.addupdate_scatter(acc_v, [d], vals)
            return 0
        lax.fori_loop(0, EPT // L, scat_body, 0)

        pltpu.sync_copy(acc_v, out_hbm.at[pl.ds(wid * NP, NP)])

    if gather:
        flat = seg_kernel(dst_pad, src_pad, u)
    else:
        flat = seg_kernel(dst_pad)
    return flat.reshape(NW, NP)


def _vt(WdT_ref, W2_ref):
    return lax.dot_general(WdT_ref[...], W2_ref[...],
                           (((1,), (1,)), ((), ())),
                           preferred_element_type=jnp.float32)


def _tc1(x_pad, W1, W2, WdT, degp):
    NP = degp.shape[1]

    def body(x_ref, W1_ref, W2_ref, WdT_ref, degp_ref, u_ref, norm_ref):
        vT = _vt(WdT_ref, W2_ref)
        wT = lax.dot_general(vT, W1_ref[...], (((1,), (1,)), ((), ())),
                             preferred_element_type=jnp.float32)
        z = jnp.sum(x_ref[...] * wT, axis=1)
        deg = jnp.sum(degp_ref[...], axis=0) + 1.0
        norm = lax.rsqrt(deg)
        norm_ref[...] = norm
        u_ref[...] = norm * z

    return pl.pallas_call(
        body,
        out_shape=[jax.ShapeDtypeStruct((NP,), jnp.float32),
                   jax.ShapeDtypeStruct((NP,), jnp.float32)],
    )(x_pad, W1, W2, WdT, degp)


def _tc2(s1p, u, norm, W2, WdT, b1r):
    NP = u.shape[0]

    def body(s1p_ref, u_ref, norm_ref, W2_ref, WdT_ref, b1r_ref, u2_ref):
        vT = _vt(WdT_ref, W2_ref)
        c1 = jnp.sum(vT * b1r_ref[...])
        s1 = jnp.sum(s1p_ref[...], axis=0)
        y1 = norm_ref[...] * (s1 + u_ref[...]) + c1
        u2_ref[...] = norm_ref[...] * y1

    return pl.pallas_call(
        body,
        out_shape=jax.ShapeDtypeStruct((NP,), jnp.float32),
    )(s1p, u, norm, W2, WdT, b1r)


def _tc3(s2p, u2, norm, WdT, b2r, bdr):
    NP = u2.shape[0]

    def body(s2p_ref, u2_ref, norm_ref, WdT_ref, b2r_ref, bdr_ref, out_ref):
        c2 = jnp.sum(WdT_ref[...] * b2r_ref[...]) + jnp.sum(bdr_ref[...])
        s2 = jnp.sum(s2p_ref[...], axis=0)
        out_ref[...] = norm_ref[...] * (s2 + u2_ref[...]) + c2

    return pl.pallas_call(
        body,
        out_shape=jax.ShapeDtypeStruct((NP,), jnp.float32),
    )(s2p, u2, norm, WdT, b2r, bdr)


def kernel(x, edge_index, W1, b1, W2, b2, Wd, bd):
    N = x.shape[0]
    E = edge_index.shape[1]

    src = edge_index[0].astype(jnp.int32)
    dst = edge_index[1].astype(jnp.int32)

    Ep = _pad_up(E, NW * L)
    if Ep > E:
        src = jnp.concatenate([src, jnp.zeros((Ep - E,), jnp.int32)])
        dst = jnp.concatenate([dst, jnp.full((Ep - E,), N, jnp.int32)])
        NP = _pad_up(N + 1, NS * 8)
    else:
        NP = _pad_up(N, NS * 8)

    x_pad = jnp.zeros((NP, x.shape[1]), jnp.float32).at[:N].set(x)
    WdT = Wd.T
    b1r = b1.reshape(1, -1)
    b2r = b2.reshape(1, -1)
    bdr = bd.reshape(1, -1)

    degp = _sc_segsum(dst, None, None, NP)
    u, norm = _tc1(x_pad, W1, W2, WdT, degp)
    s1p = _sc_segsum(dst, src, u, NP)
    u2 = _tc2(s1p, u, norm, W2, WdT, b1r)
    s2p = _sc_segsum(dst, src, u2, NP)
    o = _tc3(s2p, u2, norm, WdT, b2r, bdr)
    return o[:N, None]


# speedup vs baseline: 96.5263x; 1.0778x over previous
import functools

import jax
import jax.numpy as jnp
from jax import lax
from jax.experimental import pallas as pl
from jax.experimental.pallas import tpu as pltpu
from jax.experimental.pallas import tpu_sc as plsc

NC = 2
NS = 16
L = 16
NW = NC * NS


def _pad_up(v, m):
    return ((v + m - 1) // m) * m


def _sc_segsum(dst_pad, src_pad, u, NP):
    Ep = dst_pad.shape[0]
    EPT = Ep // NW
    gather = u is not None

    mesh = plsc.VectorSubcoreMesh(core_axis_name="c", subcore_axis_name="s",
                                  num_cores=NC)

    scratch = [
        pltpu.VMEM((EPT,), jnp.int32),
        pltpu.VMEM((NP,), jnp.float32),
    ]
    if gather:
        scratch += [
            pltpu.VMEM((EPT,), jnp.int32),
            pltpu.VMEM((NP,), jnp.float32),
        ]

    @functools.partial(
        pl.kernel,
        out_type=jax.ShapeDtypeStruct((NW * NP,), jnp.float32),
        mesh=mesh,
        scratch_types=scratch,
        compiler_params=pltpu.CompilerParams(needs_layout_passes=False),
    )
    def seg_kernel(*refs):
        if gather:
            dst_hbm, src_hbm, u_hbm, out_hbm, dst_v, acc_v, src_v, u_v = refs
        else:
            dst_hbm, out_hbm, dst_v, acc_v = refs
        cid = lax.axis_index("c")
        sid = lax.axis_index("s")
        wid = sid * NC + cid

        def zero_body(i, _):
            acc_v[pl.ds(i * L, L)] = jnp.zeros((L,), jnp.float32)
            return 0
        lax.fori_loop(0, NP // L, zero_body, 0, unroll=8)

        pltpu.sync_copy(dst_hbm.at[pl.ds(wid * EPT, EPT)], dst_v)
        if gather:
            pltpu.sync_copy(src_hbm.at[pl.ds(wid * EPT, EPT)], src_v)
            pltpu.sync_copy(u_hbm, u_v)

        ones = jnp.full((L,), 1.0, jnp.float32)

        def scat_body(i, _):
            d = dst_v[pl.ds(i * L, L)]
            if gather:
                s = src_v[pl.ds(i * L, L)]
                vals = plsc.load_gather(u_v, [s])
            else:
                vals = ones
            plsc.addupdate_scatter(acc_v, [d], vals)
            return 0
        lax.fori_loop(0, EPT // L, scat_body, 0, unroll=8)

        pltpu.sync_copy(acc_v, out_hbm.at[pl.ds(wid * NP, NP)])

    if gather:
        flat = seg_kernel(dst_pad, src_pad, u)
    else:
        flat = seg_kernel(dst_pad)
    return flat.reshape(NW, NP)


def _vt(WdT_ref, W2_ref):
    return lax.dot_general(WdT_ref[...], W2_ref[...],
                           (((1,), (1,)), ((), ())),
                           preferred_element_type=jnp.float32)


def _tc1(x_pad, W1, W2, WdT, degp):
    NP = degp.shape[1]

    def body(x_ref, W1_ref, W2_ref, WdT_ref, degp_ref, u_ref, norm_ref):
        vT = _vt(WdT_ref, W2_ref)
        wT = lax.dot_general(vT, W1_ref[...], (((1,), (1,)), ((), ())),
                             preferred_element_type=jnp.float32)
        z = jnp.sum(x_ref[...] * wT, axis=1)
        deg = jnp.sum(degp_ref[...], axis=0) + 1.0
        norm = lax.rsqrt(deg)
        norm_ref[...] = norm
        u_ref[...] = norm * z

    return pl.pallas_call(
        body,
        out_shape=[jax.ShapeDtypeStruct((NP,), jnp.float32),
                   jax.ShapeDtypeStruct((NP,), jnp.float32)],
    )(x_pad, W1, W2, WdT, degp)


def _tc2(s1p, u, norm, W2, WdT, b1r):
    NP = u.shape[0]

    def body(s1p_ref, u_ref, norm_ref, W2_ref, WdT_ref, b1r_ref, u2_ref):
        vT = _vt(WdT_ref, W2_ref)
        c1 = jnp.sum(vT * b1r_ref[...])
        s1 = jnp.sum(s1p_ref[...], axis=0)
        y1 = norm_ref[...] * (s1 + u_ref[...]) + c1
        u2_ref[...] = norm_ref[...] * y1

    return pl.pallas_call(
        body,
        out_shape=jax.ShapeDtypeStruct((NP,), jnp.float32),
    )(s1p, u, norm, W2, WdT, b1r)


def _tc3(s2p, u2, norm, WdT, b2r, bdr):
    NP = u2.shape[0]

    def body(s2p_ref, u2_ref, norm_ref, WdT_ref, b2r_ref, bdr_ref, out_ref):
        c2 = jnp.sum(WdT_ref[...] * b2r_ref[...]) + jnp.sum(bdr_ref[...])
        s2 = jnp.sum(s2p_ref[...], axis=0)
        out_ref[...] = norm_ref[...] * (s2 + u2_ref[...]) + c2

    return pl.pallas_call(
        body,
        out_shape=jax.ShapeDtypeStruct((NP,), jnp.float32),
    )(s2p, u2, norm, WdT, b2r, bdr)


def kernel(x, edge_index, W1, b1, W2, b2, Wd, bd):
    N = x.shape[0]
    E = edge_index.shape[1]

    src = edge_index[0].astype(jnp.int32)
    dst = edge_index[1].astype(jnp.int32)

    Ep = _pad_up(E, NW * L)
    if Ep > E:
        src = jnp.concatenate([src, jnp.zeros((Ep - E,), jnp.int32)])
        dst = jnp.concatenate([dst, jnp.full((Ep - E,), N, jnp.int32)])
        NP = _pad_up(N + 1, NS * 8)
    else:
        NP = _pad_up(N, NS * 8)

    x_pad = jnp.zeros((NP, x.shape[1]), jnp.float32).at[:N].set(x)
    WdT = Wd.T
    b1r = b1.reshape(1, -1)
    b2r = b2.reshape(1, -1)
    bdr = bd.reshape(1, -1)

    degp = _sc_segsum(dst, None, None, NP)
    u, norm = _tc1(x_pad, W1, W2, WdT, degp)
    s1p = _sc_segsum(dst, src, u, NP)
    u2 = _tc2(s1p, u, norm, W2, WdT, b1r)
    s2p = _sc_segsum(dst, src, u2, NP)
    o = _tc3(s2p, u2, norm, WdT, b2r, bdr)
    return o[:N, None]
